# baseline (speedup 1.0000x reference)
import functools

import jax
import jax.numpy as jnp
from jax import lax
from jax.experimental import pallas as pl
from jax.experimental.pallas import tpu as pltpu
from jax.experimental.pallas import tpu_sc as plsc

N_NODES = 10000
N_EDGES = 320000
D = 128

NC = 2
NS = 16
NW = NC * NS

K = 128
CPS = 8
SLABS = 10
E_PAD = NW * SLABS * CPS * K
RPT = 632
N_ACC = NS * RPT
DUMMY = N_NODES

_sc_mesh = plsc.VectorSubcoreMesh(core_axis_name="c", subcore_axis_name="s")


@functools.partial(
    pl.kernel,
    out_type=jax.ShapeDtypeStruct((NC, N_ACC, D), jnp.float32),
    mesh=_sc_mesh,
    scratch_types=[
        pltpu.VMEM_SHARED((N_ACC, D), jnp.float32),
        pltpu.VMEM((2, CPS, K), jnp.int32),
        pltpu.VMEM((2, K, D), jnp.float32),
        pltpu.SemaphoreType.DMA,
        pltpu.SemaphoreType.DMA,
        pltpu.SemaphoreType.DMA,
    ],
)
def _sc_scatter(x_hbm, idx_hbm, zeros_hbm, out_hbm,
                acc, idx_v, rows_v, gsem, ssem0, ssem1):
    c = lax.axis_index("c")
    s = lax.axis_index("s")
    wid = c * NS + s

    pltpu.sync_copy(zeros_hbm, acc.at[pl.ds(s * RPT, RPT)])
    plsc.subcore_barrier()

    ssems = (ssem0, ssem1)

    def slab(sl, carry):
        pltpu.sync_copy(idx_hbm.at[wid, sl], idx_v)
        descs = [None, None]
        for j in range(CPS):
            bi = j % 2
            if descs[bi] is not None:
                descs[bi].wait()
            pltpu.async_copy(x_hbm.at[idx_v.at[0, j]], rows_v.at[bi],
                             gsem).wait()
            descs[bi] = pltpu.async_copy(rows_v.at[bi], acc.at[idx_v.at[1, j]],
                                         ssems[bi], add=True)
        for bi in range(2):
            descs[bi].wait()
        return carry

    lax.fori_loop(0, SLABS, slab, 0)
    plsc.subcore_barrier()

    pltpu.sync_copy(acc.at[pl.ds(s * RPT, RPT)],
                    out_hbm.at[c].at[pl.ds(s * RPT, RPT)])


_TC_BLK = 1000


def _linear_body(pa_ref, pb_ref, w_ref, b_ref, o_ref):
    agg = pa_ref[0] + pb_ref[0]
    o_ref[...] = lax.dot_general(
        agg, w_ref[...], (((1,), (1,)), ((), ())),
        preferred_element_type=jnp.float32) + b_ref[...]


def _tc_linear(partials, w, b):
    b2 = b.reshape(1, D)
    return pl.pallas_call(
        _linear_body,
        grid=(N_NODES // _TC_BLK,),
        in_specs=[
            pl.BlockSpec((1, _TC_BLK, D), lambda i: (0, i, 0)),
            pl.BlockSpec((1, _TC_BLK, D), lambda i: (1, i, 0)),
            pl.BlockSpec((D, D), lambda i: (0, 0)),
            pl.BlockSpec((1, D), lambda i: (0, 0)),
        ],
        out_specs=pl.BlockSpec((_TC_BLK, D), lambda i: (i, 0)),
        out_shape=jax.ShapeDtypeStruct((N_NODES, D), jnp.float32),
    )(partials, partials, w, b2)


def kernel(x, edge_index, W, b):
    src = edge_index[0]
    dst = edge_index[1]
    pad = E_PAD - N_EDGES
    src_p = jnp.pad(src, (0, pad)).reshape(NW, SLABS, CPS, K)
    dst_p = jnp.pad(dst, (0, pad), constant_values=DUMMY).reshape(
        NW, SLABS, CPS, K)
    idx = jnp.stack([src_p, dst_p], axis=2)
    zeros = jnp.zeros((RPT, D), jnp.float32)
    partials = _sc_scatter(x, idx, zeros)
    return _tc_linear(partials, W, b)


# speedup vs baseline: 1.4698x; 1.4698x over previous
import functools

import jax
import jax.numpy as jnp
from jax import lax
from jax.experimental import pallas as pl
from jax.experimental.pallas import tpu as pltpu
from jax.experimental.pallas import tpu_sc as plsc

N_NODES = 10000
N_EDGES = 320000
D = 128

NC = 2
NS = 16
NW = NC * NS

K = 128
CPW = 79
E_PAD = NW * CPW * K
RPT = 632
N_ACC = NS * RPT

_sc_mesh = plsc.VectorSubcoreMesh(core_axis_name="c", subcore_axis_name="s")


@functools.partial(
    pl.kernel,
    out_type=jax.ShapeDtypeStruct((NC, N_ACC, D), jnp.float32),
    mesh=_sc_mesh,
    scratch_types=[
        pltpu.VMEM_SHARED((N_ACC, D), jnp.float32),
        pltpu.VMEM((CPW, K), jnp.int32),
        pltpu.VMEM((CPW, K), jnp.int32),
        pltpu.VMEM((K, D), jnp.float32),
        pltpu.SemaphoreType.DMA,
    ],
)
def _sc_scatter(x_hbm, src_hbm, dst_hbm, zeros_hbm, out_hbm,
                acc, src_v, dst_v, rows_v, sem):
    c = lax.axis_index("c")
    s = lax.axis_index("s")
    wid = c * NS + s

    pltpu.sync_copy(zeros_hbm, acc.at[pl.ds(s * RPT, RPT)])
    pltpu.sync_copy(src_hbm.at[wid], src_v)
    pltpu.sync_copy(dst_hbm.at[wid], dst_v)
    plsc.subcore_barrier()

    def chunk(j, carry):
        pltpu.async_copy(x_hbm.at[src_v.at[j]], rows_v, sem).wait()
        pltpu.sync_copy(rows_v, acc.at[dst_v.at[j]], add=True)
        return carry

    lax.fori_loop(0, CPW, chunk, 0)
    plsc.subcore_barrier()

    pltpu.sync_copy(acc.at[pl.ds(s * RPT, RPT)],
                    out_hbm.at[c].at[pl.ds(s * RPT, RPT)])


_TC_BLK = 1000


def _linear_body(pa_ref, pb_ref, w_ref, b_ref, o_ref):
    agg = pa_ref[0] + pb_ref[0]
    o_ref[...] = lax.dot_general(
        agg, w_ref[...], (((1,), (1,)), ((), ())),
        preferred_element_type=jnp.float32) + b_ref[...]


def _tc_linear(partials, w, b):
    b2 = b.reshape(1, D)
    return pl.pallas_call(
        _linear_body,
        grid=(N_NODES // _TC_BLK,),
        in_specs=[
            pl.BlockSpec((1, _TC_BLK, D), lambda i: (0, i, 0)),
            pl.BlockSpec((1, _TC_BLK, D), lambda i: (1, i, 0)),
            pl.BlockSpec((D, D), lambda i: (0, 0)),
            pl.BlockSpec((1, D), lambda i: (0, 0)),
        ],
        out_specs=pl.BlockSpec((_TC_BLK, D), lambda i: (i, 0)),
        out_shape=jax.ShapeDtypeStruct((N_NODES, D), jnp.float32),
    )(partials, partials, w, b2)


def kernel(x, edge_index, W, b):
    src = edge_index[0]
    dst = edge_index[1]
    pad = E_PAD - N_EDGES
    src_p = jnp.pad(src, (0, pad)).reshape(NW, CPW, K)
    dst_p = jnp.pad(dst, (0, pad), constant_values=N_NODES).reshape(NW, CPW, K)
    zeros = jnp.zeros((RPT, D), jnp.float32)
    partials = _sc_scatter(x, src_p, dst_p, zeros)
    return _tc_linear(partials, W, b)
